# baseline (speedup 1.0000x reference)
import functools

import jax
import jax.numpy as jnp
from jax import lax
from jax.experimental import pallas as pl
from jax.experimental.pallas import tpu as pltpu
from jax.experimental.pallas import tpu_sc as plsc

NC = 2
NS = 16
NW = NC * NS
CHUNK = 128
NBUF = 4
VB = 16384
LW = 128


def _tc_transpose_pack(t_view):
    D, V = t_view.shape
    grid = (V // 2 + VB - 1) // VB
    h_blocks = grid
    last_ok = (V - 1) // VB

    def body(xlo_ref, xhi_ref, o_ref):
        o_ref[...] = jnp.concatenate([xlo_ref[...].T, xhi_ref[...].T], axis=1)

    return pl.pallas_call(
        body,
        grid=(grid,),
        in_specs=[
            pl.BlockSpec((D, VB), lambda i: (0, i)),
            pl.BlockSpec((D, VB), lambda i: (0, jnp.minimum(i + h_blocks, last_ok))),
        ],
        out_specs=pl.BlockSpec((VB, 2 * D), lambda i: (i, 0)),
        out_shape=jax.ShapeDtypeStruct((grid * VB, 2 * D), jnp.float32),
    )(t_view, t_view)


def _sc_gather(table, idx):
    _, D = table.shape
    cw = idx.shape[1]
    rows = idx.shape[0] * cw * CHUNK
    mesh = plsc.VectorSubcoreMesh(core_axis_name="c", subcore_axis_name="s")

    @functools.partial(
        pl.kernel,
        mesh=mesh,
        compiler_params=pltpu.CompilerParams(use_tc_tiling_on_sc=False),
        out_type=jax.ShapeDtypeStruct((rows, D), jnp.float32),
        scratch_types=[
            pltpu.VMEM((cw, CHUNK), jnp.int32),
            pltpu.VMEM((NBUF, CHUNK, D), jnp.float32),
            pltpu.SemaphoreType.DMA((NBUF,)),
        ],
    )
    def k(table_hbm, idx_hbm, out_hbm, idx_v, bufs, sems):
        w = lax.axis_index("s") * NC + lax.axis_index("c")
        pltpu.sync_copy(idx_hbm.at[w], idx_v)
        base = w * cw * CHUNK

        for j in range(NBUF):
            pltpu.async_copy(table_hbm.at[idx_v.at[j]], bufs.at[j], sems.at[j])

        steady = cw - NBUF
        if steady > 0:
            @pl.loop(0, steady, step=NBUF)
            def _(c):
                for j in range(NBUF):
                    pltpu.make_async_copy(
                        table_hbm.at[idx_v.at[j]], bufs.at[j], sems.at[j]
                    ).wait()
                    pltpu.sync_copy(
                        bufs.at[j], out_hbm.at[pl.ds(base + (c + j) * CHUNK, CHUNK)]
                    )
                    pltpu.async_copy(
                        table_hbm.at[idx_v.at[c + j + NBUF]], bufs.at[j], sems.at[j]
                    )

        for j in range(NBUF):
            pltpu.make_async_copy(
                table_hbm.at[idx_v.at[j]], bufs.at[j], sems.at[j]
            ).wait()
            pltpu.sync_copy(
                bufs.at[j], out_hbm.at[pl.ds(base + (steady + j) * CHUNK, CHUNK)]
            )

    return k(table, idx)


def _softplus(x):
    return jnp.maximum(x, 0.0) + jnp.log1p(jnp.exp(-jnp.abs(x)))


def _tc_loss(v_rows, pv, cat_mat, pcat, sel_lo, sel_hi, B, KP1, D):
    BB = 1024
    grid = B // BB
    inv_b = 1.0 / B
    W = KP1 * LW

    def body(v_ref, pv_ref, cat_ref, pcat_ref, slo_ref, shi_ref, o_ref):
        i = pl.program_id(0)
        v2 = v_ref[...]
        pvb = pv_ref[...]
        cat = cat_ref[...]
        pc = pcat_ref[...]
        vsel = jnp.where(pvb > 0, v2[:, D:2 * D], v2[:, :D])
        vp = jnp.concatenate([vsel, vsel], axis=1)
        vt = jnp.concatenate([vp] * KP1, axis=1)
        prod = vt * cat
        dn = (((1,), (0,)), ((), ()))
        l0 = lax.dot_general(prod, slo_ref[...], dn,
                             preferred_element_type=jnp.float32)
        l1 = lax.dot_general(prod, shi_ref[...], dn,
                             preferred_element_type=jnp.float32)
        logits = jnp.where(pc > 0, l1, l0)
        col = lax.broadcasted_iota(jnp.int32, logits.shape, 1)
        signed = jnp.where(col == 0, -logits, logits)
        sp = _softplus(signed)
        blk = jnp.sum(jnp.where(col < KP1, sp, 0.0))

        @pl.when(i == 0)
        def _():
            o_ref[0] = 0.0

        o_ref[0] += blk * inv_b

    return pl.pallas_call(
        body,
        grid=(grid,),
        in_specs=[
            pl.BlockSpec((BB, LW), lambda i: (i, 0)),
            pl.BlockSpec((BB, 1), lambda i: (i, 0)),
            pl.BlockSpec((BB, W), lambda i: (i, 0)),
            pl.BlockSpec((BB, KP1), lambda i: (i, 0)),
            pl.BlockSpec((W, KP1), lambda i: (0, 0)),
            pl.BlockSpec((W, KP1), lambda i: (0, 0)),
        ],
        out_specs=pl.BlockSpec(
            (1,), lambda i: (0,), memory_space=pltpu.SMEM
        ),
        out_shape=jax.ShapeDtypeStruct((1,), jnp.float32),
    )(v_rows, pv, cat_mat, pcat, sel_lo, sel_hi)


def kernel(center, pos, neg, in_emb, out_emb):
    B = center.shape[0]
    K = neg.shape[1]
    V, D = in_emb.shape
    KP1 = K + 1
    H = V // 2

    H2 = ((H + VB - 1) // VB) * VB

    cen = center.astype(jnp.int32)
    cat = jnp.concatenate(
        [pos.astype(jnp.int32)[:, None], neg.astype(jnp.int32)], axis=1
    )
    cen_idx = jnp.where(cen < H2, cen, cen - H2).reshape(
        NW, B // NW // CHUNK, CHUNK
    )
    cat_idx = jnp.where(cat < H2, cat, cat - H2).reshape(
        NW, B * KP1 // NW // CHUNK, CHUNK
    )
    pv = (cen >= H2).astype(jnp.float32).reshape(B, 1)
    pcat = (cat >= H2).astype(jnp.float32)

    c_ar = jnp.arange(KP1 * LW)
    grp = c_ar[:, None] // LW == jnp.arange(KP1)[None, :]
    sel_lo = (grp & (c_ar[:, None] % LW < D)).astype(jnp.float32)
    sel_hi = (grp & (c_ar[:, None] % LW >= D)).astype(jnp.float32)

    out_row = _tc_transpose_pack(out_emb.T)
    cat_rows = _sc_gather(out_row, cat_idx)
    in_row = _tc_transpose_pack(in_emb.T)
    v_rows = _sc_gather(in_row, cen_idx)
    loss = _tc_loss(v_rows, pv, cat_rows.reshape(B, KP1 * LW), pcat,
                    sel_lo, sel_hi, B, KP1, D)
    return loss


# speedup vs baseline: 1.1954x; 1.1954x over previous
import functools

import jax
import jax.numpy as jnp
from jax import lax
from jax.experimental import pallas as pl
from jax.experimental.pallas import tpu as pltpu
from jax.experimental.pallas import tpu_sc as plsc

NC = 2
NS = 16
NW = NC * NS
CHUNK = 128
NBUF = 4
VB = 16384
LW = 128


def _tc_transpose_pack(t_view):
    D, V = t_view.shape
    grid = (V // 2 + VB - 1) // VB
    h_blocks = grid
    last_ok = (V - 1) // VB

    def body(xlo_ref, xhi_ref, o_ref):
        o_ref[...] = jnp.concatenate([xlo_ref[...].T, xhi_ref[...].T], axis=1)

    return pl.pallas_call(
        body,
        grid=(grid,),
        in_specs=[
            pl.BlockSpec((D, VB), lambda i: (0, i)),
            pl.BlockSpec((D, VB), lambda i: (0, jnp.minimum(i + h_blocks, last_ok))),
        ],
        out_specs=pl.BlockSpec((VB, 2 * D), lambda i: (i, 0)),
        out_shape=jax.ShapeDtypeStruct((grid * VB, 2 * D), jnp.float32),
    )(t_view, t_view)


def _sc_gather(table, idx):
    _, D = table.shape
    cw = idx.shape[1]
    rows = idx.shape[0] * cw * CHUNK
    mesh = plsc.VectorSubcoreMesh(core_axis_name="c", subcore_axis_name="s")

    @functools.partial(
        pl.kernel,
        mesh=mesh,
        compiler_params=pltpu.CompilerParams(use_tc_tiling_on_sc=False),
        out_type=jax.ShapeDtypeStruct((rows, D), jnp.float32),
        scratch_types=[
            pltpu.VMEM((cw, CHUNK), jnp.int32),
            pltpu.VMEM((NBUF, CHUNK, D), jnp.float32),
            pltpu.SemaphoreType.DMA((NBUF,)),
        ],
    )
    def k(table_hbm, idx_hbm, out_hbm, idx_v, bufs, sems):
        w = lax.axis_index("s") * NC + lax.axis_index("c")
        pltpu.sync_copy(idx_hbm.at[w], idx_v)
        base = w * cw * CHUNK

        for j in range(NBUF):
            pltpu.async_copy(table_hbm.at[idx_v.at[j]], bufs.at[j], sems.at[j])

        steady = cw - NBUF
        if steady > 0:
            @pl.loop(0, steady, step=NBUF)
            def _(c):
                for j in range(NBUF):
                    pltpu.make_async_copy(
                        table_hbm.at[idx_v.at[j]], bufs.at[j], sems.at[j]
                    ).wait()
                    pltpu.sync_copy(
                        bufs.at[j], out_hbm.at[pl.ds(base + (c + j) * CHUNK, CHUNK)]
                    )
                    pltpu.async_copy(
                        table_hbm.at[idx_v.at[c + j + NBUF]], bufs.at[j], sems.at[j]
                    )

        for j in range(NBUF):
            pltpu.make_async_copy(
                table_hbm.at[idx_v.at[j]], bufs.at[j], sems.at[j]
            ).wait()
            pltpu.sync_copy(
                bufs.at[j], out_hbm.at[pl.ds(base + (steady + j) * CHUNK, CHUNK)]
            )

    return k(table, idx)


def _sc_gather_split(table, idx, B, KP1):
    _, D = table.shape
    cw = idx.shape[1]
    tpj = B // NW // CHUNK
    mesh = plsc.VectorSubcoreMesh(core_axis_name="c", subcore_axis_name="s")

    @functools.partial(
        pl.kernel,
        mesh=mesh,
        compiler_params=pltpu.CompilerParams(use_tc_tiling_on_sc=False),
        out_type=[jax.ShapeDtypeStruct((B, D), jnp.float32) for _ in range(KP1)],
        scratch_types=[
            pltpu.VMEM((cw, CHUNK), jnp.int32),
            pltpu.VMEM((NBUF, CHUNK, D), jnp.float32),
            pltpu.SemaphoreType.DMA((NBUF,)),
        ],
    )
    def k(table_hbm, idx_hbm, *refs):
        outs = refs[:KP1]
        idx_v, bufs, sems = refs[KP1], refs[KP1 + 1], refs[KP1 + 2]
        w = lax.axis_index("s") * NC + lax.axis_index("c")
        pltpu.sync_copy(idx_hbm.at[w], idx_v)
        base = w * (B // NW)

        for j in range(NBUF):
            pltpu.async_copy(table_hbm.at[idx_v.at[j]], bufs.at[j], sems.at[j])
        for c in range(cw):
            j = c % NBUF
            pltpu.make_async_copy(
                table_hbm.at[idx_v.at[j]], bufs.at[j], sems.at[j]
            ).wait()
            pltpu.sync_copy(
                bufs.at[j],
                outs[c // tpj].at[pl.ds(base + (c % tpj) * CHUNK, CHUNK)],
            )
            if c + NBUF < cw:
                pltpu.async_copy(
                    table_hbm.at[idx_v.at[c + NBUF]], bufs.at[j], sems.at[j]
                )

    return k(table, idx)


def _softplus(x):
    return jnp.maximum(x, 0.0) + jnp.log1p(jnp.exp(-jnp.abs(x)))


def _tc_loss(v_rows, pv, cat_list, pcat, sel_lo, sel_hi, B, KP1, D):
    BB = 1024
    grid = B // BB
    inv_b = 1.0 / B
    W = KP1 * LW

    def body(v_ref, pv_ref, *rest):
        cat_refs = rest[:KP1]
        pcat_ref, slo_ref, shi_ref, o_ref = rest[KP1:]
        i = pl.program_id(0)
        v2 = v_ref[...]
        pvb = pv_ref[...]
        cat = jnp.concatenate([r[...] for r in cat_refs], axis=1)
        pc = pcat_ref[...]
        vsel = jnp.where(pvb > 0, v2[:, D:2 * D], v2[:, :D])
        vp = jnp.concatenate([vsel, vsel], axis=1)
        vt = jnp.concatenate([vp] * KP1, axis=1)
        prod = vt * cat
        dn = (((1,), (0,)), ((), ()))
        l0 = lax.dot_general(prod, slo_ref[...], dn,
                             preferred_element_type=jnp.float32)
        l1 = lax.dot_general(prod, shi_ref[...], dn,
                             preferred_element_type=jnp.float32)
        logits = jnp.where(pc > 0, l1, l0)
        col = lax.broadcasted_iota(jnp.int32, logits.shape, 1)
        signed = jnp.where(col == 0, -logits, logits)
        sp = _softplus(signed)
        blk = jnp.sum(jnp.where(col < KP1, sp, 0.0))

        @pl.when(i == 0)
        def _():
            o_ref[0] = 0.0

        o_ref[0] += blk * inv_b

    return pl.pallas_call(
        body,
        grid=(grid,),
        in_specs=[
            pl.BlockSpec((BB, LW), lambda i: (i, 0)),
            pl.BlockSpec((BB, 1), lambda i: (i, 0)),
        ] + [
            pl.BlockSpec((BB, LW), lambda i: (i, 0)) for _ in range(KP1)
        ] + [
            pl.BlockSpec((BB, KP1), lambda i: (i, 0)),
            pl.BlockSpec((W, KP1), lambda i: (0, 0)),
            pl.BlockSpec((W, KP1), lambda i: (0, 0)),
        ],
        out_specs=pl.BlockSpec(
            (1,), lambda i: (0,), memory_space=pltpu.SMEM
        ),
        out_shape=jax.ShapeDtypeStruct((1,), jnp.float32),
    )(v_rows, pv, *cat_list, pcat, sel_lo, sel_hi)


def kernel(center, pos, neg, in_emb, out_emb):
    B = center.shape[0]
    K = neg.shape[1]
    V, D = in_emb.shape
    KP1 = K + 1
    H = V // 2

    H2 = ((H + VB - 1) // VB) * VB

    cen = center.astype(jnp.int32)
    cat = jnp.concatenate(
        [pos.astype(jnp.int32)[:, None], neg.astype(jnp.int32)], axis=1
    )
    cen_idx = jnp.where(cen < H2, cen, cen - H2).reshape(
        NW, B // NW // CHUNK, CHUNK
    )
    tpj = B // NW // CHUNK
    cat_idx = (
        jnp.where(cat < H2, cat, cat - H2)
        .reshape(NW, tpj, CHUNK, KP1)
        .transpose(0, 3, 1, 2)
        .reshape(NW, KP1 * tpj, CHUNK)
    )
    pv = (cen >= H2).astype(jnp.float32).reshape(B, 1)
    pcat = (cat >= H2).astype(jnp.float32)

    c_ar = jnp.arange(KP1 * LW)
    grp = c_ar[:, None] // LW == jnp.arange(KP1)[None, :]
    sel_lo = (grp & (c_ar[:, None] % LW < D)).astype(jnp.float32)
    sel_hi = (grp & (c_ar[:, None] % LW >= D)).astype(jnp.float32)

    out_row = _tc_transpose_pack(out_emb.T)
    cat_list = _sc_gather_split(out_row, cat_idx, B, KP1)
    in_row = _tc_transpose_pack(in_emb.T)
    v_rows = _sc_gather(in_row, cen_idx)
    loss = _tc_loss(v_rows, pv, cat_list, pcat, sel_lo, sel_hi, B, KP1, D)
    return loss
